# Initial kernel scaffold; baseline (speedup 1.0000x reference)
import jax
import jax.numpy as jnp
from jax.experimental import pallas as pl


def kernel(h, pos, edge_index, h_init, Wm1, bm1, gm1, bbm1, Wm2, bm2, gm2, bbm2, Wu1, bu1, gu1, bbu1, Wp1, bp1, gp1, bbp1, Wp2, bp2, gp2, bbp2):
    raise NotImplementedError("write your pallas kernel here")


# speedup vs baseline: 2.4135x; 2.4135x over previous
import dataclasses
import functools

import jax
import jax.numpy as jnp
from jax import lax
from jax.experimental import pallas as pl
from jax.experimental.pallas import tpu as pltpu
from jax.experimental.pallas import tpu_sc as plsc

F32 = jnp.float32
I32 = jnp.int32

_N = 10000
_E = 320000
_D = 128
_NC = 2
_NS = 16
_NW = _NC * _NS
_EPW = _E // _NW
_CH = 80
_NCH = _EPW // _CH
_NG = _CH // 16

_B = 2000
_GRID = _E // _B
_B1 = 512
_GRID1 = _E // _B1

_EPS = 1e-5


def _sc_params():
    cp = pltpu.CompilerParams()
    if "needs_layout_passes" in pltpu.CompilerParams.__dataclass_fields__:
        cp = dataclasses.replace(cp, needs_layout_passes=False)
    return cp


def _invsqrt16(x):
    i = lax.bitcast_convert_type(x, I32)
    i = jnp.int32(0x5F3759DF) - lax.shift_right_logical(i, 1)
    y = lax.bitcast_convert_type(i, F32)
    for _ in range(3):
        y = y * (1.5 - 0.5 * x * y * y)
    return y


def _p0_body(h_ref, wa_ref, wb_ref, bm1_ref, td_ref, ts_ref):
    hh = h_ref[...]
    td_ref[...] = jnp.dot(hh, wa_ref[...], preferred_element_type=F32) + bm1_ref[...]
    ts_ref[...] = jnp.dot(hh, wb_ref[...], preferred_element_type=F32)


def _p0(h, wa, wb, bm1r):
    return pl.pallas_call(
        _p0_body,
        out_shape=[jax.ShapeDtypeStruct((_N, _D), F32),
                   jax.ShapeDtypeStruct((_N, _D), F32)],
    )(h, wa, wb, bm1r)


def _s1(td, ts, dst, src, px, py, pz, wd):
    mesh = plsc.VectorSubcoreMesh(core_axis_name="c", subcore_axis_name="s")

    @functools.partial(
        pl.kernel,
        out_type=jax.ShapeDtypeStruct((_E, _D), F32),
        mesh=mesh,
        compiler_params=_sc_params(),
        scratch_types=[
            pltpu.VMEM((_N,), F32),
            pltpu.VMEM((_N,), F32),
            pltpu.VMEM((_N,), F32),
            pltpu.VMEM((_D,), F32),
            pltpu.VMEM((_CH,), I32),
            pltpu.VMEM((_CH,), I32),
            pltpu.VMEM((_CH, _D), F32),
            pltpu.VMEM((_CH, _D), F32),
            pltpu.VMEM((_CH, _D), F32),
            pltpu.VMEM((_CH + 16,), F32),
            pltpu.SemaphoreType.DMA,
            pltpu.SemaphoreType.DMA,
        ],
    )
    def s1(td_hbm, ts_hbm, dst_hbm, src_hbm, px_hbm, py_hbm, pz_hbm, wd_hbm,
           zp_hbm, pxb, pyb, pzb, wdb, idxd, idxs, bufd, bufs, zbuf, dbufv,
           semd, sems):
        wid = lax.axis_index("s") * _NC + lax.axis_index("c")
        base = wid * _EPW
        pltpu.sync_copy(px_hbm, pxb)
        pltpu.sync_copy(py_hbm, pyb)
        pltpu.sync_copy(pz_hbm, pzb)
        pltpu.sync_copy(wd_hbm, wdb)

        @pl.loop(0, _NCH)
        def _(ci):
            off = base + ci * _CH
            pltpu.sync_copy(dst_hbm.at[pl.ds(off, _CH)], idxd)
            pltpu.sync_copy(src_hbm.at[pl.ds(off, _CH)], idxs)
            cpd = pltpu.async_copy(td_hbm.at[idxd], bufd, semd)
            cps = pltpu.async_copy(ts_hbm.at[idxs], bufs, sems)
            cpd.wait()
            cps.wait()

            for g in range(_NG):
                sl = pl.ds(g * 16, 16)
                id16 = idxd[sl]
                is16 = idxs[sl]
                dx = plsc.load_gather(pxb, [is16]) - plsc.load_gather(pxb, [id16])
                dy = plsc.load_gather(pyb, [is16]) - plsc.load_gather(pyb, [id16])
                dz = plsc.load_gather(pzb, [is16]) - plsc.load_gather(pzb, [id16])
                d2 = dx * dx + dy * dy + dz * dz + 1e-12
                dbufv[sl] = d2 * _invsqrt16(d2)

            @pl.loop(0, _CH)
            def _(e):
                d = dbufv[pl.ds(e, 16)][0]
                for k in range(_D // 16):
                    sl = pl.ds(k * 16, 16)
                    zbuf[e, sl] = bufd[e, sl] + bufs[e, sl] + d * wdb[sl]

            pltpu.sync_copy(zbuf, zp_hbm.at[pl.ds(off, _CH)])

    return s1(td, ts, dst, src, px, py, pz, wd)


def _p2_body(zp_ref, st_ref):
    z1 = zp_ref[...]
    st = jnp.concatenate([jnp.sum(z1, axis=0, keepdims=True),
                          jnp.sum(z1 * z1, axis=0, keepdims=True)], axis=0)

    @pl.when(pl.program_id(0) == 0)
    def _():
        st_ref[...] = jnp.zeros_like(st_ref)

    st_ref[...] += st


def _p2(zp):
    return pl.pallas_call(
        _p2_body,
        grid=(_GRID,),
        in_specs=[pl.BlockSpec((_B, _D), lambda i: (i, 0))],
        out_specs=pl.BlockSpec((2, _D), lambda i: (0, 0)),
        out_shape=jax.ShapeDtypeStruct((2, _D), F32),
    )(zp)


def _bn_coef(st_ref, g_ref, b_ref):
    s = st_ref[0:1, :]
    sq = st_ref[1:2, :]
    m = s * (1.0 / _E)
    v = sq * (1.0 / _E) - m * m
    a = g_ref[...] * lax.rsqrt(v + _EPS)
    c = b_ref[...] - m * a
    return a, c


def _p3_body(zp_ref, st1_ref, g_ref, b_ref, w2_ref, b2_ref, z2_ref, st2_ref):
    a, c = _bn_coef(st1_ref, g_ref, b_ref)
    a1 = jnp.maximum(a * zp_ref[...] + c, 0.0)
    z2 = jnp.dot(a1, w2_ref[...], preferred_element_type=F32) + b2_ref[...]
    z2_ref[...] = z2
    st = jnp.concatenate([jnp.sum(z2, axis=0, keepdims=True),
                          jnp.sum(z2 * z2, axis=0, keepdims=True)], axis=0)

    @pl.when(pl.program_id(0) == 0)
    def _():
        st2_ref[...] = jnp.zeros_like(st2_ref)

    st2_ref[...] += st


def _p3(zp, st1, gm1r, bbm1r, wm2, bm2r):
    return pl.pallas_call(
        _p3_body,
        grid=(_GRID,),
        in_specs=[pl.BlockSpec((_B, _D), lambda i: (i, 0)),
                  pl.BlockSpec((2, _D), lambda i: (0, 0)),
                  pl.BlockSpec((1, _D), lambda i: (0, 0)),
                  pl.BlockSpec((1, _D), lambda i: (0, 0)),
                  pl.BlockSpec((_D, _D), lambda i: (0, 0)),
                  pl.BlockSpec((1, _D), lambda i: (0, 0))],
        out_specs=[pl.BlockSpec((_B, _D), lambda i: (i, 0)),
                   pl.BlockSpec((2, _D), lambda i: (0, 0))],
        out_shape=[jax.ShapeDtypeStruct((_E, _D), F32),
                   jax.ShapeDtypeStruct((2, _D), F32)],
    )(zp, st1, gm1r, bbm1r, wm2, bm2r)


def _p4_body(z2_ref, st2_ref, g_ref, b_ref, wp1_ref, bp1_ref,
             msg_ref, z3_ref, st3_ref):
    a, c = _bn_coef(st2_ref, g_ref, b_ref)
    msg = jnp.maximum(a * z2_ref[...] + c, 0.0)
    msg_ref[...] = msg
    z3 = jnp.dot(msg, wp1_ref[...], preferred_element_type=F32) + bp1_ref[...]
    z3_ref[...] = z3
    st = jnp.concatenate([jnp.sum(z3, axis=0, keepdims=True),
                          jnp.sum(z3 * z3, axis=0, keepdims=True)], axis=0)

    @pl.when(pl.program_id(0) == 0)
    def _():
        st3_ref[...] = jnp.zeros_like(st3_ref)

    st3_ref[...] += st


def _p4(z2, st2, gm2r, bbm2r, wp1, bp1r):
    return pl.pallas_call(
        _p4_body,
        grid=(_GRID,),
        in_specs=[pl.BlockSpec((_B, _D), lambda i: (i, 0)),
                  pl.BlockSpec((2, _D), lambda i: (0, 0)),
                  pl.BlockSpec((1, _D), lambda i: (0, 0)),
                  pl.BlockSpec((1, _D), lambda i: (0, 0)),
                  pl.BlockSpec((_D, _D), lambda i: (0, 0)),
                  pl.BlockSpec((1, _D), lambda i: (0, 0))],
        out_specs=[pl.BlockSpec((_B, _D), lambda i: (i, 0)),
                   pl.BlockSpec((_B, _D), lambda i: (i, 0)),
                   pl.BlockSpec((2, _D), lambda i: (0, 0))],
        out_shape=[jax.ShapeDtypeStruct((_E, _D), F32),
                   jax.ShapeDtypeStruct((_E, _D), F32),
                   jax.ShapeDtypeStruct((2, _D), F32)],
    )(z2, st2, gm2r, bbm2r, wp1, bp1r)


def _p5_body(z3_ref, st3_ref, g_ref, b_ref, wp2t_ref, bp2_ref, z4_ref, st4_ref):
    a, c = _bn_coef(st3_ref, g_ref, b_ref)
    a3 = jnp.maximum(a * z3_ref[...] + c, 0.0)
    z4row = lax.dot_general(wp2t_ref[...], a3, (((1,), (1,)), ((), ())),
                            preferred_element_type=F32) + bp2_ref[0, 0]
    z4_ref[...] = z4row[0]
    st = jnp.concatenate([jnp.sum(z4row, axis=1, keepdims=True),
                          jnp.sum(z4row * z4row, axis=1, keepdims=True)], axis=0)

    @pl.when(pl.program_id(0) == 0)
    def _():
        st4_ref[...] = jnp.zeros_like(st4_ref)

    st4_ref[...] += st


def _p5(z3, st3, gp1r, bbp1r, wp2t, bp2r):
    return pl.pallas_call(
        _p5_body,
        grid=(_GRID1,),
        in_specs=[pl.BlockSpec((_B1, _D), lambda i: (i, 0)),
                  pl.BlockSpec((2, _D), lambda i: (0, 0)),
                  pl.BlockSpec((1, _D), lambda i: (0, 0)),
                  pl.BlockSpec((1, _D), lambda i: (0, 0)),
                  pl.BlockSpec((1, _D), lambda i: (0, 0)),
                  pl.BlockSpec((1, 1), lambda i: (0, 0))],
        out_specs=[pl.BlockSpec((_B1,), lambda i: (i,)),
                   pl.BlockSpec((2, 1), lambda i: (0, 0))],
        out_shape=[jax.ShapeDtypeStruct((_E,), F32),
                   jax.ShapeDtypeStruct((2, 1), F32)],
    )(z3, st3, gp1r, bbp1r, wp2t, bp2r)


def _p5c_body(z4_ref, st4_ref, g_ref, b_ref, s_ref):
    m = st4_ref[0, 0] * (1.0 / _E)
    v = st4_ref[1, 0] * (1.0 / _E) - m * m
    a = g_ref[0, 0] * lax.rsqrt(v + _EPS)
    c = b_ref[0, 0] - m * a
    s_ref[...] = jnp.maximum(a * z4_ref[...] + c, 0.0)


def _p5c(z4, st4, gp2r, bbp2r):
    return pl.pallas_call(
        _p5c_body,
        grid=(_GRID1,),
        in_specs=[pl.BlockSpec((_B1,), lambda i: (i,)),
                  pl.BlockSpec((2, 1), lambda i: (0, 0)),
                  pl.BlockSpec((1, 1), lambda i: (0, 0)),
                  pl.BlockSpec((1, 1), lambda i: (0, 0))],
        out_specs=pl.BlockSpec((_B1,), lambda i: (i,)),
        out_shape=jax.ShapeDtypeStruct((_E,), F32),
    )(z4, st4, gp2r, bbp2r)


def _s2(msg, dst, z128):
    mesh = plsc.VectorSubcoreMesh(core_axis_name="c", subcore_axis_name="s")
    rows = (_N // _NS) // 8 * 8

    @functools.partial(
        pl.kernel,
        out_type=jax.ShapeDtypeStruct((_NC * _N, _D), F32),
        mesh=mesh,
        compiler_params=_sc_params(),
        scratch_types=[
            pltpu.VMEM((_CH,), I32),
            pltpu.VMEM((_CH, _D), F32),
            pltpu.VMEM_SHARED((_N, _D), F32),
        ],
    )
    def s2(msg_hbm, dst_hbm, z128_hbm, hag_hbm, idxd, mbuf, hacc):
        cid = lax.axis_index("c")
        sid = lax.axis_index("s")
        wid = sid * _NC + cid

        @pl.when(sid == 0)
        def _():
            pltpu.sync_copy(z128_hbm, hacc)

        plsc.subcore_barrier()

        @pl.loop(0, _NCH)
        def _(ci):
            off = wid * _EPW + ci * _CH
            pltpu.sync_copy(dst_hbm.at[pl.ds(off, _CH)], idxd)
            pltpu.sync_copy(msg_hbm.at[pl.ds(off, _CH)], mbuf)
            pltpu.sync_copy(mbuf, hacc.at[idxd], add=True)

        plsc.subcore_barrier()
        r0 = sid * rows
        pltpu.sync_copy(hacc.at[pl.ds(r0, rows)],
                        hag_hbm.at[pl.ds(cid * _N + r0, rows)])

        @pl.when(sid == 0)
        def _():
            t0 = _NS * rows
            tn = _N - _NS * rows
            pltpu.sync_copy(hacc.at[pl.ds(t0, tn)],
                            hag_hbm.at[pl.ds(cid * _N + t0, tn)])

    return s2(msg, dst, z128)


def _s2b(s, dst, src, px, py, pz, z128):
    mesh = plsc.VectorSubcoreMesh(core_axis_name="c", subcore_axis_name="s")
    rows = (_N // _NS) // 8 * 8

    @functools.partial(
        pl.kernel,
        out_type=jax.ShapeDtypeStruct((_NC * _N, _D), F32),
        mesh=mesh,
        compiler_params=_sc_params(),
        scratch_types=[
            pltpu.VMEM((_N,), F32),
            pltpu.VMEM((_N,), F32),
            pltpu.VMEM((_N,), F32),
            pltpu.VMEM((_CH,), I32),
            pltpu.VMEM((_CH,), I32),
            pltpu.VMEM((_CH,), F32),
            pltpu.VMEM((_CH, _D), F32),
            pltpu.VMEM_SHARED((_N, _D), F32),
        ],
    )
    def s2b(s_hbm, dst_hbm, src_hbm, px_hbm, py_hbm, pz_hbm, z128_hbm, dag_hbm,
            pxb, pyb, pzb, idxd, idxs, sbuf, dsbuf, dacc):
        cid = lax.axis_index("c")
        sid = lax.axis_index("s")
        wid = sid * _NC + cid
        pltpu.sync_copy(px_hbm, pxb)
        pltpu.sync_copy(py_hbm, pyb)
        pltpu.sync_copy(pz_hbm, pzb)

        lane = lax.iota(I32, 16)
        pat = jnp.where(lane == 3, 1.0, 0.0).astype(F32)
        zv = jnp.zeros((16,), F32)

        @pl.loop(0, _CH)
        def _(rr):
            dsbuf[rr, pl.ds(0, 16)] = pat
            for k in range(1, _D // 16):
                dsbuf[rr, pl.ds(k * 16, 16)] = zv

        @pl.when(sid == 0)
        def _():
            pltpu.sync_copy(z128_hbm, dacc)

        plsc.subcore_barrier()

        @pl.loop(0, _NCH)
        def _(ci):
            off = wid * _EPW + ci * _CH
            pltpu.sync_copy(dst_hbm.at[pl.ds(off, _CH)], idxd)
            pltpu.sync_copy(src_hbm.at[pl.ds(off, _CH)], idxs)
            pltpu.sync_copy(s_hbm.at[pl.ds(off, _CH)], sbuf)

            for g in range(_NG):
                sl = pl.ds(g * 16, 16)
                id16 = idxd[sl]
                is16 = idxs[sl]
                s16 = sbuf[sl]
                dxs = (plsc.load_gather(pxb, [is16])
                       - plsc.load_gather(pxb, [id16])) * s16
                dys = (plsc.load_gather(pyb, [is16])
                       - plsc.load_gather(pyb, [id16])) * s16
                dzs = (plsc.load_gather(pzb, [is16])
                       - plsc.load_gather(pzb, [id16])) * s16
                rows16 = lane + (g * 16)
                plsc.store_scatter(dsbuf, [rows16, jnp.full((16,), 0, I32)], dxs)
                plsc.store_scatter(dsbuf, [rows16, jnp.full((16,), 1, I32)], dys)
                plsc.store_scatter(dsbuf, [rows16, jnp.full((16,), 2, I32)], dzs)

            pltpu.sync_copy(dsbuf, dacc.at[idxd], add=True)

        plsc.subcore_barrier()
        r0 = sid * rows
        pltpu.sync_copy(dacc.at[pl.ds(r0, rows)],
                        dag_hbm.at[pl.ds(cid * _N + r0, rows)])

        @pl.when(sid == 0)
        def _():
            t0 = _NS * rows
            tn = _N - _NS * rows
            pltpu.sync_copy(dacc.at[pl.ds(t0, tn)],
                            dag_hbm.at[pl.ds(cid * _N + t0, tn)])

    return s2b(s, dst, src, px, py, pz, z128)


def _p6a_body(h_ref, hag_ref, wua_ref, wub_ref, bu_ref, g_ref, b_ref, out_ref):
    hh = h_ref[...]
    hs = hag_ref[0:_N, :] + hag_ref[_N:2 * _N, :]
    zu = (jnp.dot(hh, wua_ref[...], preferred_element_type=F32)
          + jnp.dot(hs, wub_ref[...], preferred_element_type=F32)
          + bu_ref[...])
    m = jnp.mean(zu, axis=0, keepdims=True)
    v = jnp.mean(zu * zu, axis=0, keepdims=True) - m * m
    a = g_ref[...] * lax.rsqrt(v + _EPS)
    c = b_ref[...] - m * a
    out_ref[...] = hh + jnp.maximum(a * zu + c, 0.0)


def _p6a(h, hag, wua, wub, bu1r, gu1r, bbu1r):
    return pl.pallas_call(
        _p6a_body,
        out_shape=jax.ShapeDtypeStruct((_N, _D), F32),
    )(h, hag, wua, wub, bu1r, gu1r, bbu1r)


def _p6b_body(p16_ref, dag_ref, out_ref):
    dsum = dag_ref[0:_N, 0:16] + dag_ref[_N:2 * _N, 0:16]
    cnt = dsum[:, 3:4]
    inv = 1.0 / jnp.maximum(cnt, 1.0)
    lane = lax.broadcasted_iota(I32, (_N, 16), 1)
    xa = dsum * inv * jnp.where(lane < 3, 1.0, 0.0).astype(F32)
    out_ref[...] = p16_ref[...] + xa


def _p6b(pos16, dag):
    return pl.pallas_call(
        _p6b_body,
        out_shape=jax.ShapeDtypeStruct((_N, 16), F32),
    )(pos16, dag)


def kernel(h, pos, edge_index, h_init,
           Wm1, bm1, gm1, bbm1, Wm2, bm2, gm2, bbm2,
           Wu1, bu1, gu1, bbu1,
           Wp1, bp1, gp1, bbp1, Wp2, bp2, gp2, bbp2):
    src = edge_index[0]
    dst = edge_index[1]
    px = pos[:, 0]
    py = pos[:, 1]
    pz = pos[:, 2]
    pos16 = jnp.concatenate([pos, jnp.zeros((_N, 13), F32)], axis=1)

    wa = Wm1[:_D]
    wb = Wm1[_D:2 * _D]
    wd = Wm1[2 * _D]
    wp2t = Wp2.reshape(1, _D)
    r = lambda x: x.reshape(1, -1)
    r11 = lambda x: x.reshape(1, 1)

    td, ts = _p0(h, wa, wb, r(bm1))
    zp = _s1(td, ts, dst, src, px, py, pz, wd)
    st1 = _p2(zp)
    z2, st2 = _p3(zp, st1, r(gm1), r(bbm1), Wm2, r(bm2))
    msg, z3, st3 = _p4(z2, st2, r(gm2), r(bbm2), Wp1, r(bp1))
    z4, st4 = _p5(z3, st3, r(gp1), r(bbp1), wp2t, r11(bp2))
    s = _p5c(z4, st4, r11(gp2), r11(bbp2))

    z128 = jnp.zeros((_N, _D), F32)
    hag = _s2(msg, dst, z128)
    dag = _s2b(s, dst, src, px, py, pz, z128)

    out_h = _p6a(h, hag, Wu1[:_D], Wu1[_D:], r(bu1), r(gu1), r(bbu1))
    outp16 = _p6b(pos16, dag)
    return out_h, outp16[:, :3]
